# Initial kernel scaffold; baseline (speedup 1.0000x reference)
import jax
import jax.numpy as jnp
from jax.experimental import pallas as pl


def kernel(x, edge_index, edge_type, W_I1, W_O1, W_R1, rel1, b1, W_I2, W_O2, W_R2, rel2, b2):
    raise NotImplementedError("write your pallas kernel here")


# speedup vs baseline: 6.8210x; 6.8210x over previous
import functools

import jax
import jax.numpy as jnp
from jax import lax
from jax.experimental import pallas as pl
from jax.experimental.pallas import tpu as pltpu
from jax.experimental.pallas import tpu_sc as plsc

N = 10000
E = 320000
D = 128
R = 200

NC = 2
NS = 16
NW = NC * NS
EPW = E // NW
K = 80
NCH = EPW // K
RSTRIDE = 624
RSPAN = 640

_mesh = plsc.VectorSubcoreMesh(core_axis_name="c", subcore_axis_name="s")


def _sc_agg_body(x_hbm, src_hbm, dst_hbm, et_hbm, nrel_hbm, zero_hbm, out_hbm,
                 src_v, dst_v, et_v, xrows_v, rrows_v, agg_sh,
                 sem_i, sem_x, sem_r):
    c = lax.axis_index("c")
    s = lax.axis_index("s")
    wid = c * NS + s

    pltpu.sync_copy(zero_hbm.at[pl.ds(s * RSTRIDE, RSPAN)],
                    agg_sh.at[pl.ds(s * RSTRIDE, RSPAN)])
    plsc.subcore_barrier()

    def chunk(i, carry):
        ci1 = pltpu.async_copy(src_hbm.at[wid, i], src_v, sem_i)
        ci2 = pltpu.async_copy(dst_hbm.at[wid, i], dst_v, sem_i)
        ci3 = pltpu.async_copy(et_hbm.at[wid, i], et_v, sem_i)
        ci1.wait()
        ci2.wait()
        ci3.wait()
        cpx = pltpu.async_copy(x_hbm.at[src_v.at[0]], xrows_v, sem_x)
        cpr = pltpu.async_copy(nrel_hbm.at[et_v.at[0]], rrows_v, sem_r)
        cpx.wait()
        cpr.wait()
        pltpu.sync_copy(xrows_v, agg_sh.at[dst_v.at[0]], add=True)
        pltpu.sync_copy(rrows_v, agg_sh.at[dst_v.at[0]], add=True)
        return carry

    lax.fori_loop(0, NCH, chunk, 0)
    plsc.subcore_barrier()
    pltpu.sync_copy(agg_sh.at[pl.ds(s * RSTRIDE, RSPAN)],
                    out_hbm.at[pl.ds(c * N + s * RSTRIDE, RSPAN)])


_sc_agg = pl.kernel(
    _sc_agg_body,
    out_type=jax.ShapeDtypeStruct((2 * N, D), jnp.float32),
    mesh=_mesh,
    scratch_types=[
        pltpu.VMEM((1, K), jnp.int32),
        pltpu.VMEM((1, K), jnp.int32),
        pltpu.VMEM((1, K), jnp.int32),
        pltpu.VMEM((K, D), jnp.float32),
        pltpu.VMEM((K, D), jnp.float32),
        pltpu.VMEM_SHARED((N, D), jnp.float32),
        pltpu.SemaphoreType.DMA,
        pltpu.SemaphoreType.DMA,
        pltpu.SemaphoreType.DMA,
    ],
)


def _dense_body(relu, a0_ref, a1_ref, x_ref, wi_ref, wo_ref, r0_ref, b_ref, o_ref):
    xb = x_ref[...]
    m = a0_ref[...] + a1_ref[...] + xb
    dn = (((1,), (1,)), ((), ()))
    t = lax.dot_general(m, wi_ref[...], dn, preferred_element_type=jnp.float32)
    t = t + lax.dot_general(xb, wo_ref[...], dn, preferred_element_type=jnp.float32)
    shift = lax.dot_general(r0_ref[...], wi_ref[...], dn,
                            preferred_element_type=jnp.float32)
    t = t + (b_ref[...] - shift)
    o_ref[...] = jnp.maximum(t, 0.0) if relu else t


BLK = 400
GRID = N // BLK


def _dense(agg2, x, w_i, w_o, rel0, b, relu):
    return pl.pallas_call(
        functools.partial(_dense_body, relu),
        grid=(GRID,),
        in_specs=[
            pl.BlockSpec((BLK, D), lambda i: (i, 0)),
            pl.BlockSpec((BLK, D), lambda i: (i + GRID, 0)),
            pl.BlockSpec((BLK, D), lambda i: (i, 0)),
            pl.BlockSpec((D, D), lambda i: (0, 0)),
            pl.BlockSpec((D, D), lambda i: (0, 0)),
            pl.BlockSpec((1, D), lambda i: (0, 0)),
            pl.BlockSpec((1, D), lambda i: (0, 0)),
        ],
        out_specs=pl.BlockSpec((BLK, D), lambda i: (i, 0)),
        out_shape=jax.ShapeDtypeStruct((N, D), jnp.float32),
    )(agg2, agg2, x, w_i, w_o, rel0, b)


def kernel(x, edge_index, edge_type, W_I1, W_O1, W_R1, rel1, b1,
           W_I2, W_O2, W_R2, rel2, b2):
    src = edge_index[0].reshape(NW, NCH, 1, K)
    dst = edge_index[1].reshape(NW, NCH, 1, K)
    et = edge_type.reshape(NW, NCH, 1, K)
    zeros = jnp.zeros((N, D), jnp.float32)

    agg1 = _sc_agg(x, src, dst, et, -rel1, zeros)
    h = _dense(agg1, x, W_I1, W_O1, rel1[0:1], b1.reshape(1, D), relu=True)
    agg2 = _sc_agg(h, src, dst, et, -rel2, zeros)
    out = _dense(agg2, h, W_I2, W_O2, rel2[0:1], b2.reshape(1, D), relu=False)
    return out


# speedup vs baseline: 9.5088x; 1.3941x over previous
import functools

import jax
import jax.numpy as jnp
from jax import lax
from jax.experimental import pallas as pl
from jax.experimental.pallas import tpu as pltpu
from jax.experimental.pallas import tpu_sc as plsc

N = 10000
E = 320000
D = 128
R = 200

NC = 2
NS = 16
NW = NC * NS
EPW = E // NW
K = 80
NCH = EPW // K
NPAIR = (NCH - 1) // 2
RSTRIDE = 624
RSPAN = 640

_mesh = plsc.VectorSubcoreMesh(core_axis_name="c", subcore_axis_name="s")


def _sc_agg_body(x_hbm, se_hbm, dst_hbm, nrel_hbm, zero_hbm, out_hbm,
                 se0, se1, d0, d1, xb0, rb0, xb1, rb1, agg_sh,
                 semse0, semse1, semd0, semd1, semg0, semg1, sems0, sems1):
    c = lax.axis_index("c")
    s = lax.axis_index("s")
    wid = c * NS + s

    pltpu.sync_copy(zero_hbm.at[pl.ds(s * RSTRIDE, RSPAN)],
                    agg_sh.at[pl.ds(s * RSTRIDE, RSPAN)])
    plsc.subcore_barrier()

    seb = (se0, se1)
    db = (d0, d1)
    xb = (xb0, xb1)
    rb = (rb0, rb1)
    semse = (semse0, semse1)
    semd = (semd0, semd1)
    semg = (semg0, semg1)
    sems = (sems0, sems1)

    def fetch_se(i, p):
        pltpu.async_copy(se_hbm.at[wid, i], seb[p], semse[p])

    def fetch_dst(i, p):
        pltpu.async_copy(dst_hbm.at[wid, i], db[p], semd[p])

    def issue_gather(p):
        pltpu.async_copy(x_hbm.at[seb[p].at[0, 0]], xb[p], semg[p])
        pltpu.async_copy(nrel_hbm.at[seb[p].at[1, 0]], rb[p], semg[p])

    def issue_scatter(p):
        pltpu.async_copy(xb[p], agg_sh.at[db[p].at[0]], sems[p], add=True)
        pltpu.async_copy(rb[p], agg_sh.at[db[p].at[0]], sems[p], add=True)

    def drain(sem, ref):
        pltpu.make_async_copy(x_hbm.at[pl.ds(0, K)], ref, sem).wait()

    def drain_se(p):
        pltpu.make_async_copy(se_hbm.at[0, 0], seb[p], semse[p]).wait()

    def drain_dst(p):
        pltpu.make_async_copy(dst_hbm.at[0, 0], db[p], semd[p]).wait()

    def wait_gather(p):
        drain(semg[p], xb[p])
        drain(semg[p], rb[p])

    def wait_scatter(p):
        drain(sems[p], xb[p])
        drain(sems[p], rb[p])

    fetch_se(0, 0)
    fetch_se(1, 1)
    fetch_dst(0, 0)
    drain_se(0)
    issue_gather(0)

    def pair(t, carry):
        a = 2 * t
        wait_gather(0)
        fetch_se(a + 2, 0)

        @pl.when(t > 0)
        def _():
            wait_scatter(1)
        fetch_dst(a + 1, 1)
        drain_se(1)
        issue_gather(1)
        drain_dst(0)
        issue_scatter(0)

        wait_gather(1)

        @pl.when(t < NPAIR - 1)
        def _():
            fetch_se(a + 3, 1)
        wait_scatter(0)
        fetch_dst(a + 2, 0)
        drain_se(0)
        issue_gather(0)
        drain_dst(1)
        issue_scatter(1)
        return carry

    lax.fori_loop(0, NPAIR, pair, 0)
    wait_gather(0)
    wait_scatter(1)
    drain_dst(0)
    issue_scatter(0)
    wait_scatter(0)

    plsc.subcore_barrier()
    pltpu.sync_copy(agg_sh.at[pl.ds(s * RSTRIDE, RSPAN)],
                    out_hbm.at[pl.ds(c * N + s * RSTRIDE, RSPAN)])


_sc_agg = pl.kernel(
    _sc_agg_body,
    out_type=jax.ShapeDtypeStruct((2 * N, D), jnp.float32),
    mesh=_mesh,
    scratch_types=[
        pltpu.VMEM((2, 1, K), jnp.int32),
        pltpu.VMEM((2, 1, K), jnp.int32),
        pltpu.VMEM((1, K), jnp.int32),
        pltpu.VMEM((1, K), jnp.int32),
        pltpu.VMEM((K, D), jnp.float32),
        pltpu.VMEM((K, D), jnp.float32),
        pltpu.VMEM((K, D), jnp.float32),
        pltpu.VMEM((K, D), jnp.float32),
        pltpu.VMEM_SHARED((N, D), jnp.float32),
        pltpu.SemaphoreType.DMA,
        pltpu.SemaphoreType.DMA,
        pltpu.SemaphoreType.DMA,
        pltpu.SemaphoreType.DMA,
        pltpu.SemaphoreType.DMA,
        pltpu.SemaphoreType.DMA,
        pltpu.SemaphoreType.DMA,
        pltpu.SemaphoreType.DMA,
    ],
)


def _dense_body(relu, a0_ref, a1_ref, x_ref, wi_ref, wo_ref, r0_ref, b_ref, o_ref):
    xb = x_ref[...]
    m = a0_ref[...] + a1_ref[...] + xb
    dn = (((1,), (1,)), ((), ()))
    t = lax.dot_general(m, wi_ref[...], dn, preferred_element_type=jnp.float32)
    t = t + lax.dot_general(xb, wo_ref[...], dn, preferred_element_type=jnp.float32)
    shift = lax.dot_general(r0_ref[...], wi_ref[...], dn,
                            preferred_element_type=jnp.float32)
    t = t + (b_ref[...] - shift)
    o_ref[...] = jnp.maximum(t, 0.0) if relu else t


BLK = 400
GRID = N // BLK


def _dense(agg2, x, w_i, w_o, rel0, b, relu):
    return pl.pallas_call(
        functools.partial(_dense_body, relu),
        grid=(GRID,),
        in_specs=[
            pl.BlockSpec((BLK, D), lambda i: (i, 0)),
            pl.BlockSpec((BLK, D), lambda i: (i + GRID, 0)),
            pl.BlockSpec((BLK, D), lambda i: (i, 0)),
            pl.BlockSpec((D, D), lambda i: (0, 0)),
            pl.BlockSpec((D, D), lambda i: (0, 0)),
            pl.BlockSpec((1, D), lambda i: (0, 0)),
            pl.BlockSpec((1, D), lambda i: (0, 0)),
        ],
        out_specs=pl.BlockSpec((BLK, D), lambda i: (i, 0)),
        out_shape=jax.ShapeDtypeStruct((N, D), jnp.float32),
    )(agg2, agg2, x, w_i, w_o, rel0, b)


def kernel(x, edge_index, edge_type, W_I1, W_O1, W_R1, rel1, b1,
           W_I2, W_O2, W_R2, rel2, b2):
    src = edge_index[0].reshape(NW, NCH, 1, K)
    dst = edge_index[1].reshape(NW, NCH, 1, K)
    et = edge_type.reshape(NW, NCH, 1, K)
    se = jnp.stack([src, et], axis=2)
    zeros = jnp.zeros((N, D), jnp.float32)

    agg1 = _sc_agg(x, se, dst, -rel1, zeros)
    h = _dense(agg1, x, W_I1, W_O1, rel1[0:1], b1.reshape(1, D), relu=True)
    agg2 = _sc_agg(h, se, dst, -rel2, zeros)
    out = _dense(agg2, h, W_I2, W_O2, rel2[0:1], b2.reshape(1, D), relu=False)
    return out
